# speedup vs baseline: 3.3443x; 3.3443x over previous
import functools

import jax
import jax.numpy as jnp
from jax import lax
from jax.experimental import pallas as pl
from jax.experimental.pallas import tpu as pltpu, tpu_sc as plsc

_info = plsc.get_sparse_core_info()
_NC, _NS = _info.num_cores, _info.num_subcores
_NW = _NC * _NS

_B = 4096 * 50
_D = 128
_BPW = _B // _NW
_C = 400
_S = _BPW // _C


@functools.partial(
    pl.kernel,
    mesh=plsc.VectorSubcoreMesh(core_axis_name="c", subcore_axis_name="s"),
    out_type=jax.ShapeDtypeStruct((_B, _D), jnp.float32),
    scratch_types=[
        pltpu.VMEM((_BPW,), jnp.int32),
        pltpu.VMEM((_C, _D), jnp.float32),
        pltpu.VMEM((_C, _D), jnp.float32),
        pltpu.SemaphoreType.DMA,
        pltpu.SemaphoreType.DMA,
    ],
)
def _lookup(table_hbm, idx_hbm, out_hbm, idx_v, rows0, rows1, sem0, sem1):
    wid = lax.axis_index("s") * _NC + lax.axis_index("c")
    base = wid * _BPW
    pltpu.sync_copy(idx_hbm.at[pl.ds(base, _BPW)], idx_v)

    rows = [rows0, rows1]
    sems = [sem0, sem1]
    cps = [
        pltpu.async_copy(table_hbm.at[idx_v.at[pl.ds(0, _C)]], rows[0], sems[0]),
        pltpu.async_copy(table_hbm.at[idx_v.at[pl.ds(_C, _C)]], rows[1], sems[1]),
    ]
    for t in range(_S):
        b = t % 2
        cps[b].wait()
        pltpu.sync_copy(rows[b], out_hbm.at[pl.ds(base + t * _C, _C)])
        if t + 2 < _S:
            cps[b] = pltpu.async_copy(
                table_hbm.at[idx_v.at[pl.ds((t + 2) * _C, _C)]], rows[b], sems[b]
            )


def kernel(inputs, embedding_weights):
    idx = inputs.reshape(-1).astype(jnp.int32)
    out = _lookup(embedding_weights, idx)
    return out.reshape(inputs.shape[0], inputs.shape[1], _D)
